# baseline (speedup 1.0000x reference)
import functools

import jax
import jax.numpy as jnp
from jax import lax
from jax.experimental import pallas as pl
from jax.experimental.pallas import tpu as pltpu
from jax.experimental.pallas import tpu_sc as plsc

T = 8192
H = 2048
K = 2
E = 64
TK = T * K
NC = 2
NS = 16
NW = NC * NS
SPT = TK // NS
RPW = TK // NW
CH = 16
NCHUNK = RPW // CH
NBUF = 3
OFF_PAD = 80
IROW = 128
NIROW = SPT // IROW


def _sc_body(ids_hbm, x_hbm, disp_hbm, tpe_hbm, off_hbm,
             ids_v, dest_v, vals_v, histg_v, tot_v, off_v,
             idxb_v, buf_v, cnt_s, hist_sp, srcsorted_sp, *sems):
    gsems = sems[:NBUF]
    osems = sems[NBUF:]
    cid = lax.axis_index("c")
    sid = lax.axis_index("s")
    gwid = sid * NC + cid

    my_base_slot = sid * SPT
    _scope = jax.named_scope("phA_hist")
    _scope.__enter__()
    pltpu.sync_copy(ids_hbm.at[pl.ds(my_base_slot, SPT)], ids_v)

    zeros16 = jnp.zeros((16,), jnp.int32)
    ii16 = lax.iota(jnp.int32, 16)

    for e in range(E):
        cnt_s[e] = jnp.int32(0)

    def _hist_group(g, carry):
        v = ids_v[pl.ds(g * 16, 16)]
        for l in range(16):
            e = v[l]
            cnt_s[e] = cnt_s[e] + 1
        return carry

    lax.fori_loop(0, SPT // 16, _hist_group, 0)
    _scope.__exit__(None, None, None)
    _scope = jax.named_scope("phA_merge")
    _scope.__enter__()

    for j in range(E // 16):
        vh = jnp.zeros((16,), jnp.int32)
        for l in range(16):
            vh = jnp.where(ii16 == l, cnt_s[j * 16 + l], vh)
        tot_v[pl.ds(j * 16, 16)] = vh
    pltpu.sync_copy(tot_v, hist_sp.at[pl.ds(sid * E, E)])
    plsc.subcore_barrier()
    pltpu.sync_copy(hist_sp, histg_v)

    carry = jnp.int32(0)
    for j in range(E // 16):
        tot_j = zeros16
        below_j = zeros16
        for sp in range(NS):
            row = histg_v[pl.ds(sp * E + j * 16, 16)]
            tot_j = tot_j + row
            below_j = below_j + row * (jnp.int32(sp) < sid).astype(jnp.int32)
        inc = plsc.cumsum(tot_j)
        excl = inc - tot_j + carry
        wb = excl + below_j
        carry = carry + jnp.sum(tot_j)
        tot_v[pl.ds(j * 16, 16)] = tot_j
        off_v[pl.ds(j * 16, 16)] = excl
        for l in range(16):
            cnt_s[j * 16 + l] = wb[l]

    off_v[pl.ds(E, 16)] = jnp.where(ii16 == 0, jnp.int32(TK), 0)

    @pl.when(jnp.logical_and(cid == 0, sid == 0))
    def _write_aux():
        pltpu.sync_copy(tot_v, tpe_hbm)
        pltpu.sync_copy(off_v, off_hbm)

    _scope.__exit__(None, None, None)
    _scope = jax.named_scope("phA_rank")
    _scope.__enter__()
    for j in range(NIROW):
        vrow = vals_v.at[j]
        for l in range(IROW // 16):
            vrow[pl.ds(l * 16, 16)] = (my_base_slot + j * IROW + l * 16 + ii16) // K

    for j in range(NIROW):
        drow = dest_v.at[j]

        def _rank_group(g, carry, j=j, drow=drow):
            v = ids_v[pl.ds(j * IROW + g * 16, 16)]
            dvec = jnp.zeros((16,), jnp.int32)
            for l in range(16):
                e = v[l]
                d = cnt_s[e]
                cnt_s[e] = d + 1
                dvec = jnp.where(ii16 == l, d, dvec)
            drow[pl.ds(g * 16, 16)] = dvec
            return carry

        lax.fori_loop(0, IROW // 16, _rank_group, 0)

    for j in range(NIROW):
        pltpu.sync_copy(vals_v.at[j], srcsorted_sp.at[dest_v.at[j]])
    plsc.subcore_barrier()

    _scope.__exit__(None, None, None)
    _scope = jax.named_scope("phB_gather")
    _scope.__enter__()
    base_row = gwid * RPW
    pltpu.sync_copy(srcsorted_sp.at[pl.ds(base_row, RPW)], idxb_v)

    def _start_gather(c, b):
        src0 = (gwid % 16) * RPW + c * CH
        return pltpu.async_copy(
            x_hbm.at[pl.ds(src0, CH)], buf_v.at[b], gsems[b])

    g_h = [None] * NBUF
    o_h = [None] * NBUF
    for c in range(NBUF - 1):
        g_h[c] = _start_gather(c, c)
    for c in range(NCHUNK):
        b = c % NBUF
        g_h[b].wait()
        n = c + NBUF - 1
        if n < NCHUNK:
            bn = n % NBUF
            if o_h[bn] is not None:
                o_h[bn].wait()
                o_h[bn] = None
            g_h[bn] = _start_gather(n, bn)
        if o_h[b] is not None:
            o_h[b].wait()
        o_h[b] = pltpu.async_copy(
            buf_v.at[b], disp_hbm.at[pl.ds(base_row + c * CH, CH)], osems[b])
    for b in range(NBUF):
        if o_h[b] is not None:
            o_h[b].wait()
    _scope.__exit__(None, None, None)


_sc_call = functools.partial(
    pl.kernel,
    mesh=plsc.VectorSubcoreMesh(core_axis_name="c", subcore_axis_name="s"),
    compiler_params=pltpu.CompilerParams(needs_layout_passes=False),
    out_type=[
        jax.ShapeDtypeStruct((TK, H), jnp.float32),
        jax.ShapeDtypeStruct((E,), jnp.int32),
        jax.ShapeDtypeStruct((OFF_PAD,), jnp.int32),
    ],
    scratch_types=[
        pltpu.VMEM((SPT,), jnp.int32),
        pltpu.VMEM((NIROW, IROW), jnp.int32),
        pltpu.VMEM((NIROW, IROW), jnp.int32),
        pltpu.VMEM((NS * E,), jnp.int32),
        pltpu.VMEM((E,), jnp.int32),
        pltpu.VMEM((OFF_PAD,), jnp.int32),
        pltpu.VMEM((RPW,), jnp.int32),
        pltpu.VMEM((NBUF, CH, H), jnp.float32),
        pltpu.SMEM((E,), jnp.int32),
        pltpu.VMEM_SHARED((NS * E,), jnp.int32),
        pltpu.VMEM_SHARED((TK,), jnp.int32),
    ] + [pltpu.SemaphoreType.DMA] * (2 * NBUF),
)(_sc_body)


def _combined_body(x_ref, w_ref, o_ref):
    w = w_ref[...]
    o_ref[...] = x_ref[...] * jnp.sum(w, axis=1, keepdims=True)


_combined_call = pl.pallas_call(
    _combined_body,
    grid=(T // 512,),
    in_specs=[
        pl.BlockSpec((512, H), lambda i: (i, 0)),
        pl.BlockSpec((512, K), lambda i: (i, 0)),
    ],
    out_specs=pl.BlockSpec((512, H), lambda i: (i, 0)),
    out_shape=jax.ShapeDtypeStruct((T, H), jnp.float32),
)


def kernel(x, topk_weights, topk_indices):
    flat_e = topk_indices.reshape(-1)
    dispatched, tokens_per_expert, off_pad = _sc_call(flat_e, x)
    combined = _combined_call(x, topk_weights)
    offsets = off_pad[: E + 1]
    return combined, dispatched, tokens_per_expert, offsets


# speedup vs baseline: 1.1350x; 1.1350x over previous
import functools

import jax
import jax.numpy as jnp
from jax import lax
from jax.experimental import pallas as pl
from jax.experimental.pallas import tpu as pltpu
from jax.experimental.pallas import tpu_sc as plsc

T = 8192
H = 2048
K = 2
E = 64
TK = T * K
NC = 2
NS = 16
NW = NC * NS
SPT = TK // NS
TPT = SPT // K
TPW = TPT // NC
CH = 16
NCHUNK = TPW // CH
NBUF = 3
OFF_PAD = 80


def _sc_body(ids_hbm, x_hbm, disp_hbm, tpe_hbm, off_hbm,
             ids_v, deste_v, desto_v, histg_v, tot_v, off_v,
             buf_v, cnt_s, hist_sp, *sems):
    gsems = sems[:NBUF]
    osems = sems[NBUF:]
    cid = lax.axis_index("c")
    sid = lax.axis_index("s")

    my_base_slot = sid * SPT
    _scope = jax.named_scope("phA_hist")
    _scope.__enter__()
    pltpu.sync_copy(ids_hbm.at[pl.ds(my_base_slot, SPT)], ids_v)

    zeros16 = jnp.zeros((16,), jnp.int32)
    ii16 = lax.iota(jnp.int32, 16)

    for e in range(E):
        cnt_s[e] = jnp.int32(0)

    def _hist_group(g, carry):
        v = ids_v[pl.ds(g * 16, 16)]
        for l in range(16):
            e = v[l]
            cnt_s[e] = cnt_s[e] + 1
        return carry

    lax.fori_loop(0, SPT // 16, _hist_group, 0)
    _scope.__exit__(None, None, None)
    _scope = jax.named_scope("phA_merge")
    _scope.__enter__()

    for j in range(E // 16):
        vh = jnp.zeros((16,), jnp.int32)
        for l in range(16):
            vh = jnp.where(ii16 == l, cnt_s[j * 16 + l], vh)
        tot_v[pl.ds(j * 16, 16)] = vh
    pltpu.sync_copy(tot_v, hist_sp.at[pl.ds(sid * E, E)])
    plsc.subcore_barrier()
    pltpu.sync_copy(hist_sp, histg_v)

    carry = jnp.int32(0)
    for j in range(E // 16):
        tot_j = zeros16
        below_j = zeros16
        for sp in range(NS):
            row = histg_v[pl.ds(sp * E + j * 16, 16)]
            tot_j = tot_j + row
            below_j = below_j + row * (jnp.int32(sp) < sid).astype(jnp.int32)
        inc = plsc.cumsum(tot_j)
        excl = inc - tot_j + carry
        wb = excl + below_j
        carry = carry + jnp.sum(tot_j)
        tot_v[pl.ds(j * 16, 16)] = tot_j
        off_v[pl.ds(j * 16, 16)] = excl
        for l in range(16):
            cnt_s[j * 16 + l] = wb[l]

    off_v[pl.ds(E, 16)] = jnp.where(ii16 == 0, jnp.int32(TK), 0)

    @pl.when(jnp.logical_and(cid == 0, sid == 0))
    def _write_aux():
        pltpu.sync_copy(tot_v, tpe_hbm)
        pltpu.sync_copy(off_v, off_hbm)

    _scope.__exit__(None, None, None)
    _scope = jax.named_scope("phA_rank")
    _scope.__enter__()

    def _rank_group(g, carry):
        v1 = ids_v[pl.ds(g * 32, 16)]
        v2 = ids_v[pl.ds(g * 32 + 16, 16)]
        de = jnp.zeros((16,), jnp.int32)
        do = jnp.zeros((16,), jnp.int32)
        for l in range(32):
            e = v1[l] if l < 16 else v2[l - 16]
            d = cnt_s[e]
            cnt_s[e] = d + 1
            if l % 2 == 0:
                de = jnp.where(ii16 == l // 2, d, de)
            else:
                do = jnp.where(ii16 == l // 2, d, do)
        deste_v[pl.ds(g * 16, 16)] = de
        desto_v[pl.ds(g * 16, 16)] = do
        return carry

    lax.fori_loop(0, TPT // 16, _rank_group, 0)

    _scope.__exit__(None, None, None)
    _scope = jax.named_scope("phB_scatter")
    _scope.__enter__()
    tok0 = sid * TPT + cid * TPW
    loc0 = cid * TPW

    def _start_read(c, b):
        return pltpu.async_copy(
            x_hbm.at[pl.ds(tok0 + c * CH, CH)], buf_v.at[b], gsems[b])

    g_h = [None] * NBUF
    o_h = [None] * NBUF
    for c in range(NBUF - 1):
        g_h[c] = _start_read(c, c)
    for c in range(NCHUNK):
        b = c % NBUF
        g_h[b].wait()
        idx_e = deste_v[pl.ds(loc0 + c * CH, CH)]
        idx_o = desto_v[pl.ds(loc0 + c * CH, CH)]
        o_h[b] = (
            pltpu.async_copy(buf_v.at[b], disp_hbm.at[idx_e], osems[b]),
            pltpu.async_copy(buf_v.at[b], disp_hbm.at[idx_o], osems[b]),
        )
        n = c + NBUF - 1
        if n < NCHUNK:
            bn = n % NBUF
            if o_h[bn] is not None:
                for h in o_h[bn]:
                    h.wait()
                o_h[bn] = None
            g_h[bn] = _start_read(n, bn)
    for b in range(NBUF):
        if o_h[b] is not None:
            for h in o_h[b]:
                h.wait()
    _scope.__exit__(None, None, None)


_sc_call = functools.partial(
    pl.kernel,
    mesh=plsc.VectorSubcoreMesh(core_axis_name="c", subcore_axis_name="s"),
    compiler_params=pltpu.CompilerParams(needs_layout_passes=False),
    out_type=[
        jax.ShapeDtypeStruct((TK, H), jnp.float32),
        jax.ShapeDtypeStruct((E,), jnp.int32),
        jax.ShapeDtypeStruct((OFF_PAD,), jnp.int32),
    ],
    scratch_types=[
        pltpu.VMEM((SPT,), jnp.int32),
        pltpu.VMEM((TPT,), jnp.int32),
        pltpu.VMEM((TPT,), jnp.int32),
        pltpu.VMEM((NS * E,), jnp.int32),
        pltpu.VMEM((E,), jnp.int32),
        pltpu.VMEM((OFF_PAD,), jnp.int32),
        pltpu.VMEM((NBUF, CH, H), jnp.float32),
        pltpu.SMEM((E,), jnp.int32),
        pltpu.VMEM_SHARED((NS * E,), jnp.int32),
    ] + [pltpu.SemaphoreType.DMA] * (2 * NBUF),
)(_sc_body)


def _combined_body(x_ref, w_ref, o_ref):
    w = w_ref[...]
    o_ref[...] = x_ref[...] * jnp.sum(w, axis=1, keepdims=True)


_combined_call = pl.pallas_call(
    _combined_body,
    grid=(T // 512,),
    in_specs=[
        pl.BlockSpec((512, H), lambda i: (i, 0)),
        pl.BlockSpec((512, K), lambda i: (i, 0)),
    ],
    out_specs=pl.BlockSpec((512, H), lambda i: (i, 0)),
    out_shape=jax.ShapeDtypeStruct((T, H), jnp.float32),
)


def kernel(x, topk_weights, topk_indices):
    flat_e = topk_indices.reshape(-1)
    dispatched, tokens_per_expert, off_pad = _sc_call(flat_e, x)
    combined = _combined_call(x, topk_weights)
    offsets = off_pad[: E + 1]
    return combined, dispatched, tokens_per_expert, offsets


# speedup vs baseline: 1.1468x; 1.0104x over previous
import functools

import jax
import jax.numpy as jnp
from jax import lax
from jax.experimental import pallas as pl
from jax.experimental.pallas import tpu as pltpu
from jax.experimental.pallas import tpu_sc as plsc

T = 8192
H = 2048
K = 2
E = 64
TK = T * K
NC = 2
NS = 16
NW = NC * NS
SPT = TK // NS
TPT = SPT // K
TPW = TPT // NC
CH = 16
NCHUNK = TPW // CH
NBUF = 3
OFF_PAD = 80


def _sc_body(ids_hbm, x_hbm, disp_hbm, tpe_hbm, off_hbm,
             ids_v, deste_v, desto_v, histg_v, tot_v, off_v,
             buf_v, cnt_s, hist_sp, *sems):
    gsems = sems[:NBUF]
    osems = sems[NBUF:]
    cid = lax.axis_index("c")
    sid = lax.axis_index("s")

    tok0 = sid * TPT + cid * TPW

    def _start_read(c, b):
        return pltpu.async_copy(
            x_hbm.at[pl.ds(tok0 + c * CH, CH)], buf_v.at[b], gsems[b])

    g_h = [None] * NBUF
    for c in range(NBUF - 1):
        g_h[c] = _start_read(c, c)

    my_base_slot = sid * SPT
    _scope = jax.named_scope("phA_hist")
    _scope.__enter__()
    pltpu.sync_copy(ids_hbm.at[pl.ds(my_base_slot, SPT)], ids_v)

    zeros16 = jnp.zeros((16,), jnp.int32)
    ii16 = lax.iota(jnp.int32, 16)

    for e in range(E):
        cnt_s[e] = jnp.int32(0)

    def _hist_group(g, carry):
        v = ids_v[pl.ds(g * 16, 16)]
        for l in range(16):
            e = v[l]
            cnt_s[e] = cnt_s[e] + 1
        return carry

    lax.fori_loop(0, SPT // 16, _hist_group, 0)
    _scope.__exit__(None, None, None)
    _scope = jax.named_scope("phA_merge")
    _scope.__enter__()

    for j in range(E // 16):
        vh = jnp.zeros((16,), jnp.int32)
        for l in range(16):
            vh = jnp.where(ii16 == l, cnt_s[j * 16 + l], vh)
        tot_v[pl.ds(j * 16, 16)] = vh
    pltpu.sync_copy(tot_v, hist_sp.at[pl.ds(sid * E, E)])
    plsc.subcore_barrier()
    pltpu.sync_copy(hist_sp, histg_v)

    carry = jnp.int32(0)
    for j in range(E // 16):
        tot_j = zeros16
        below_j = zeros16
        for sp in range(NS):
            row = histg_v[pl.ds(sp * E + j * 16, 16)]
            tot_j = tot_j + row
            below_j = below_j + row * (jnp.int32(sp) < sid).astype(jnp.int32)
        inc = plsc.cumsum(tot_j)
        excl = inc - tot_j + carry
        wb = excl + below_j
        carry = carry + jnp.sum(tot_j)
        tot_v[pl.ds(j * 16, 16)] = tot_j
        off_v[pl.ds(j * 16, 16)] = excl
        for l in range(16):
            cnt_s[j * 16 + l] = wb[l]

    off_v[pl.ds(E, 16)] = jnp.where(ii16 == 0, jnp.int32(TK), 0)

    @pl.when(jnp.logical_and(cid == 0, sid == 0))
    def _write_aux():
        pltpu.sync_copy(tot_v, tpe_hbm)
        pltpu.sync_copy(off_v, off_hbm)

    _scope.__exit__(None, None, None)
    _scope = jax.named_scope("phA_rank")
    _scope.__enter__()

    def _rank_group(g, carry):
        v1 = ids_v[pl.ds(g * 32, 16)]
        v2 = ids_v[pl.ds(g * 32 + 16, 16)]
        de = jnp.zeros((16,), jnp.int32)
        do = jnp.zeros((16,), jnp.int32)
        for l in range(32):
            e = v1[l] if l < 16 else v2[l - 16]
            d = cnt_s[e]
            cnt_s[e] = d + 1
            if l % 2 == 0:
                de = jnp.where(ii16 == l // 2, d, de)
            else:
                do = jnp.where(ii16 == l // 2, d, do)
        deste_v[pl.ds(g * 16, 16)] = de
        desto_v[pl.ds(g * 16, 16)] = do
        return carry

    lax.fori_loop(0, TPT // 16, _rank_group, 0)

    _scope.__exit__(None, None, None)
    _scope = jax.named_scope("phB_scatter")
    _scope.__enter__()
    loc0 = cid * TPW
    o_h = [None] * NBUF
    for c in range(NCHUNK):
        b = c % NBUF
        g_h[b].wait()
        idx_e = deste_v[pl.ds(loc0 + c * CH, CH)]
        idx_o = desto_v[pl.ds(loc0 + c * CH, CH)]
        o_h[b] = (
            pltpu.async_copy(buf_v.at[b], disp_hbm.at[idx_e], osems[b]),
            pltpu.async_copy(buf_v.at[b], disp_hbm.at[idx_o], osems[b]),
        )
        n = c + NBUF - 1
        if n < NCHUNK:
            bn = n % NBUF
            if o_h[bn] is not None:
                for h in o_h[bn]:
                    h.wait()
                o_h[bn] = None
            g_h[bn] = _start_read(n, bn)
    for b in range(NBUF):
        if o_h[b] is not None:
            for h in o_h[b]:
                h.wait()
    _scope.__exit__(None, None, None)


_sc_call = functools.partial(
    pl.kernel,
    mesh=plsc.VectorSubcoreMesh(core_axis_name="c", subcore_axis_name="s"),
    compiler_params=pltpu.CompilerParams(needs_layout_passes=False),
    out_type=[
        jax.ShapeDtypeStruct((TK, H), jnp.float32),
        jax.ShapeDtypeStruct((E,), jnp.int32),
        jax.ShapeDtypeStruct((OFF_PAD,), jnp.int32),
    ],
    scratch_types=[
        pltpu.VMEM((SPT,), jnp.int32),
        pltpu.VMEM((TPT,), jnp.int32),
        pltpu.VMEM((TPT,), jnp.int32),
        pltpu.VMEM((NS * E,), jnp.int32),
        pltpu.VMEM((E,), jnp.int32),
        pltpu.VMEM((OFF_PAD,), jnp.int32),
        pltpu.VMEM((NBUF, CH, H), jnp.float32),
        pltpu.SMEM((E,), jnp.int32),
        pltpu.VMEM_SHARED((NS * E,), jnp.int32),
    ] + [pltpu.SemaphoreType.DMA] * (2 * NBUF),
)(_sc_body)


def _combined_body(x_ref, w_ref, o_ref):
    w = w_ref[...]
    o_ref[...] = x_ref[...] * jnp.sum(w, axis=1, keepdims=True)


_combined_call = pl.pallas_call(
    _combined_body,
    grid=(T // 512,),
    in_specs=[
        pl.BlockSpec((512, H), lambda i: (i, 0)),
        pl.BlockSpec((512, K), lambda i: (i, 0)),
    ],
    out_specs=pl.BlockSpec((512, H), lambda i: (i, 0)),
    out_shape=jax.ShapeDtypeStruct((T, H), jnp.float32),
)


def kernel(x, topk_weights, topk_indices):
    flat_e = topk_indices.reshape(-1)
    dispatched, tokens_per_expert, off_pad = _sc_call(flat_e, x)
    combined = _combined_call(x, topk_weights)
    offsets = off_pad[: E + 1]
    return combined, dispatched, tokens_per_expert, offsets
